# baseline (speedup 1.0000x reference)
import functools

import jax
import jax.numpy as jnp
from jax import lax
from jax.experimental import pallas as pl
from jax.experimental.pallas import tpu as pltpu
from jax.experimental.pallas import tpu_sc as plsc

_B = 64
_NTOT = 1024
_DIM = 256
_LANES = 16
_CPI = 16384
_NW = 32
_BPW = _B // _NW
_NG = 32
_GSZ = (_BPW * _CPI) // _NG
_GV = _GSZ // _LANES
_NSLOT = 4
_LAG = 1


def _body(img_hbm, idx_hbm, out_hbm, idxv, bbuf, *rest):
    wid = lax.axis_index("s") * 2 + lax.axis_index("c")
    base_off = wid * _BPW * _CPI

    ibufs = rest[:_NSLOT]
    dbufs = rest[_NSLOT:2 * _NSLOT]
    gsems = rest[2 * _NSLOT:3 * _NSLOT]
    ssems = rest[3 * _NSLOT:4 * _NSLOT]

    pltpu.sync_copy(idx_hbm, idxv)

    lane = lax.iota(jnp.int32, _LANES)
    lhalf = lane >> 3
    lg8 = (lane & 7) * 8

    def bld(i16, carry):
        s = idxv[pl.ds(i16 * 16, 16)]
        bbuf[pl.ds(i16 * 16, 16)] = (
            ((s >> 5) << 9) + (((s >> 3) & 3) << 6) + (s & 7))
        return carry

    lax.fori_loop(0, _NTOT // 16, bld, 0)

    pats = [(k & 3) * 2 + lhalf for k in range(8)]
    advs = [lg8 + ((k >> 2) & 1) * 256 for k in range(8)]

    def build_group(t, ibuf):
        img_off = (t >> 4) * _CPI + base_off
        advt = [a + img_off for a in advs]

        def octet(j, carry):
            b8 = ((t & 15) * 8 + j) * 8
            for k in range(8):
                bi = plsc.load_gather(bbuf, [b8 + pats[k]])
                ibuf[pl.ds((j * 8 + k) * 16, 16)] = bi + advt[k]
            return carry

        lax.fori_loop(0, _GV // 8, octet, 0)

    def start_gather(t, k):
        build_group(t, ibufs[k])
        pltpu.async_copy(img_hbm.at[ibufs[k]], dbufs[k], gsems[k])

    def wait_gather(k):
        pltpu.make_async_copy(img_hbm.at[pl.ds(0, _GSZ)], dbufs[k],
                              gsems[k]).wait()

    def out_dst(t):
        return out_hbm.at[pl.ds(base_off + t * _GSZ, _GSZ)]

    def start_scatter(t, k):
        pltpu.async_copy(dbufs[k], out_dst(t), ssems[k])

    def wait_scatter(t, k):
        pltpu.make_async_copy(dbufs[k], out_dst(t), ssems[k]).wait()

    ahead = _NSLOT - _LAG
    for t in range(ahead):
        start_gather(t, t)
    for t in range(_LAG):
        wait_gather(t)
        start_scatter(t, t)
        start_gather(t + ahead, t + ahead)

    n_steady = _NG - ahead - _LAG
    assert n_steady % _NSLOT == 0

    def step(t8, carry):
        for k in range(_NSLOT):
            t = _LAG + t8 * _NSLOT + k
            sl = (_LAG + k) % _NSLOT
            wait_gather(sl)
            start_scatter(t, sl)
            fr = (sl + _NSLOT - _LAG) % _NSLOT
            wait_scatter(t - _LAG, fr)
            start_gather(t + ahead, fr)
        return carry

    lax.fori_loop(0, n_steady // _NSLOT, step, 0)

    for t in range(_NG - ahead, _NG):
        sl = t % _NSLOT
        wait_gather(sl)
        start_scatter(t, sl)
        wait_scatter(t - _LAG, (t - _LAG) % _NSLOT)
    for t in range(_NG - _LAG, _NG):
        wait_scatter(t, t % _NSLOT)


@jax.jit
def _sc_gather(images2d, patch_indices):
    mesh = plsc.VectorSubcoreMesh(core_axis_name="c", subcore_axis_name="s")
    return pl.kernel(
        _body,
        out_type=jax.ShapeDtypeStruct((_B * _CPI, _LANES), jnp.float32),
        mesh=mesh,
        compiler_params=pltpu.CompilerParams(
            needs_layout_passes=False, use_tc_tiling_on_sc=False),
        scratch_types=(
            [pltpu.VMEM((_NTOT,), jnp.int32),
             pltpu.VMEM((_NTOT,), jnp.int32)]
            + [pltpu.VMEM((_GSZ,), jnp.int32) for _ in range(_NSLOT)]
            + [pltpu.VMEM((_GSZ, _LANES), jnp.float32) for _ in range(_NSLOT)]
            + [pltpu.SemaphoreType.DMA for _ in range(2 * _NSLOT)]
        ),
    )(images2d, patch_indices)


def kernel(images, patch_indices, patch_mask):
    x = (images.reshape(_B, 64, 8, 4, 128)
         .transpose(0, 1, 3, 2, 4)
         .reshape(_B * _CPI, _LANES))
    y = _sc_gather(x, patch_indices)
    out = (y.reshape(_B, 128, 2, 8, 128)
           .transpose(0, 1, 3, 2, 4)
           .reshape(_B, _NTOT, _DIM))
    return lax.cond(
        jnp.all(patch_mask),
        lambda o: o,
        lambda o: o * patch_mask.astype(o.dtype)[None],
        out,
    )


# speedup vs baseline: 1.0186x; 1.0186x over previous
import functools

import jax
import jax.numpy as jnp
from jax import lax
from jax.experimental import pallas as pl
from jax.experimental.pallas import tpu as pltpu
from jax.experimental.pallas import tpu_sc as plsc

_B = 64
_NTOT = 1024
_DIM = 256
_LANES = 16
_CPI = 16384
_NW = 32
_BPW = _B // _NW
_NG = 64
_GSZ = (_BPW * _CPI) // _NG
_GV = _GSZ // _LANES
_NSLOT = 8
_LAG = 3


def _body(img_hbm, idx_hbm, out_hbm, idxv, bbuf, *rest):
    wid = lax.axis_index("s") * 2 + lax.axis_index("c")
    base_off = wid * _BPW * _CPI

    ibufs = rest[:_NSLOT]
    dbufs = rest[_NSLOT:2 * _NSLOT]
    gsems = rest[2 * _NSLOT:3 * _NSLOT]
    ssems = rest[3 * _NSLOT:4 * _NSLOT]

    pltpu.sync_copy(idx_hbm, idxv)

    lane = lax.iota(jnp.int32, _LANES)
    lhalf = lane >> 3
    lg8 = (lane & 7) * 8

    def bld(i16, carry):
        s = idxv[pl.ds(i16 * 16, 16)]
        bbuf[pl.ds(i16 * 16, 16)] = (
            ((s >> 5) << 9) + (((s >> 3) & 3) << 6) + (s & 7))
        return carry

    lax.fori_loop(0, _NTOT // 16, bld, 0)

    def build_group(t, ibuf):
        def one(v, carry):
            dv = t * _GV + v
            dvm = dv & 1023
            i_scal = ((dvm >> 3) << 3) + (dvm & 3) * 2
            bi = plsc.load_gather(bbuf, [i_scal + lhalf])
            addend = (((dvm >> 2) & 1) << 8) + ((dv >> 10) * _CPI + base_off)
            ibuf[pl.ds(v * 16, 16)] = bi + lg8 + addend
            return carry

        lax.fori_loop(0, _GV, one, 0)

    def start_gather(t, k):
        build_group(t, ibufs[k])
        pltpu.async_copy(img_hbm.at[ibufs[k]], dbufs[k], gsems[k])

    def wait_gather(k):
        pltpu.make_async_copy(img_hbm.at[pl.ds(0, _GSZ)], dbufs[k],
                              gsems[k]).wait()

    def out_dst(t):
        return out_hbm.at[pl.ds(base_off + t * _GSZ, _GSZ)]

    def start_scatter(t, k):
        pltpu.async_copy(dbufs[k], out_dst(t), ssems[k])

    def wait_scatter(t, k):
        pltpu.make_async_copy(dbufs[k], out_dst(t), ssems[k]).wait()

    ahead = _NSLOT - _LAG
    for t in range(ahead):
        start_gather(t, t)
    for t in range(_LAG):
        wait_gather(t)
        start_scatter(t, t)
        start_gather(t + ahead, t + ahead)

    n_steady = _NG - ahead - _LAG
    assert n_steady % _NSLOT == 0

    def step(t8, carry):
        for k in range(_NSLOT):
            t = _LAG + t8 * _NSLOT + k
            sl = (_LAG + k) % _NSLOT
            wait_gather(sl)
            start_scatter(t, sl)
            fr = (sl + _NSLOT - _LAG) % _NSLOT
            wait_scatter(t - _LAG, fr)
            start_gather(t + ahead, fr)
        return carry

    lax.fori_loop(0, n_steady // _NSLOT, step, 0)

    for t in range(_NG - ahead, _NG):
        sl = t % _NSLOT
        wait_gather(sl)
        start_scatter(t, sl)
        wait_scatter(t - _LAG, (t - _LAG) % _NSLOT)
    for t in range(_NG - _LAG, _NG):
        wait_scatter(t, t % _NSLOT)


@jax.jit
def _sc_gather(images2d, patch_indices):
    mesh = plsc.VectorSubcoreMesh(core_axis_name="c", subcore_axis_name="s")
    return pl.kernel(
        _body,
        out_type=jax.ShapeDtypeStruct((_B * _CPI, _LANES), jnp.float32),
        mesh=mesh,
        compiler_params=pltpu.CompilerParams(
            needs_layout_passes=False, use_tc_tiling_on_sc=False),
        scratch_types=(
            [pltpu.VMEM((_NTOT,), jnp.int32),
             pltpu.VMEM((_NTOT,), jnp.int32)]
            + [pltpu.VMEM((_GSZ,), jnp.int32) for _ in range(_NSLOT)]
            + [pltpu.VMEM((_GSZ, _LANES), jnp.float32) for _ in range(_NSLOT)]
            + [pltpu.SemaphoreType.DMA for _ in range(2 * _NSLOT)]
        ),
    )(images2d, patch_indices)


def kernel(images, patch_indices, patch_mask):
    x = (images.reshape(_B, 64, 8, 4, 128)
         .transpose(0, 1, 3, 2, 4)
         .reshape(_B * _CPI, _LANES))
    y = _sc_gather(x, patch_indices)
    out = (y.reshape(_B, 128, 2, 8, 128)
           .transpose(0, 1, 3, 2, 4)
           .reshape(_B, _NTOT, _DIM))
    return lax.cond(
        jnp.all(patch_mask),
        lambda o: o,
        lambda o: o * patch_mask.astype(o.dtype)[None],
        out,
    )
